# baseline (speedup 1.0000x reference)
import functools

import jax
import jax.numpy as jnp
from jax import lax
from jax.experimental import pallas as pl
from jax.experimental.pallas import tpu as pltpu
from jax.experimental.pallas import tpu_sc as plsc

_BATCH = 16384
_D = 64
_TPB = 8
_NC = 2
_NS = 16
_NW = _NC * _NS
_BPW = _BATCH // _NW
_L = 16
_CH = 32
_NCH = _BPW // _CH


def _sc_body(users_h, pos_h, neg_h, ut_h, it_h,
             xp_out, reg_out,
             sdx_u, sdx_p, sdx_n,
             ru, rp, rn, parts, racc, sem):
    wid = lax.axis_index("s") * _NC + lax.axis_index("c")
    base = wid * _BPW

    for j in range(_BPW // 128):
        sl = pl.ds(base + j * 128, 128)
        pltpu.sync_copy(users_h.at[sl], sdx_u.at[j])
        pltpu.sync_copy(pos_h.at[sl], sdx_p.at[j])
        pltpu.sync_copy(neg_h.at[sl], sdx_n.at[j])

    zero = jnp.zeros((_L,), jnp.float32)

    def round_body(c, sacc):
        jr = lax.shift_right_logical(c, 2)
        orow = jnp.bitwise_and(c, 3) * _CH
        subs = []
        copies = []
        for g in range(_CH // _L):
            goff = orow + g * _L
            vu = sdx_u[jr, pl.ds(goff, _L)]
            vp = sdx_p[jr, pl.ds(goff, _L)]
            vn = sdx_n[jr, pl.ds(goff, _L)]
            for r in range(_L):
                i = g * _L + r
                eu, ep, en = vu[r], vp[r], vn[r]
                subs.append((jnp.bitwise_and(eu, 7), jnp.bitwise_and(ep, 7),
                             jnp.bitwise_and(en, 7)))
                copies.append(pltpu.async_copy(
                    ut_h.at[lax.shift_right_logical(eu, 3)], ru.at[i], sem))
                copies.append(pltpu.async_copy(
                    it_h.at[lax.shift_right_logical(ep, 3)], rp.at[i], sem))
                copies.append(pltpu.async_copy(
                    it_h.at[lax.shift_right_logical(en, 3)], rn.at[i], sem))
        for cp_ in copies:
            cp_.wait()
        for i in range(_CH):
            su, sp, sn = subs[i]
            pv = zero
            for kk in range(_D // _L):
                sl = pl.ds(kk * _L, _L)
                u = ru[i, su, sl]
                p = rp[i, sp, sl]
                n = rn[i, sn, sl]
                pv = pv + u * (p - n)
                sacc = sacc + u * u + p * p + n * n
            parts[pl.ds((c * _CH + i) * _L, _L)] = pv
        return sacc

    sacc = lax.fori_loop(0, _NCH, round_body, zero)
    racc[...] = sacc
    pltpu.sync_copy(parts, xp_out.at[pl.ds(base * _L, _BPW * _L)])
    pltpu.sync_copy(racc, reg_out.at[pl.ds(wid * _L, _L)])


_sc_gather_dot = functools.partial(
    pl.kernel,
    mesh=plsc.VectorSubcoreMesh(core_axis_name="c", subcore_axis_name="s"),
    out_type=[
        jax.ShapeDtypeStruct((_BATCH * _L,), jnp.float32),
        jax.ShapeDtypeStruct((_NW * _L,), jnp.float32),
    ],
    scratch_types=[
        pltpu.VMEM((_BPW // 128, 128), jnp.int32),
        pltpu.VMEM((_BPW // 128, 128), jnp.int32),
        pltpu.VMEM((_BPW // 128, 128), jnp.int32),
        pltpu.VMEM((_CH, _TPB, _D), jnp.float32),
        pltpu.VMEM((_CH, _TPB, _D), jnp.float32),
        pltpu.VMEM((_CH, _TPB, _D), jnp.float32),
        pltpu.VMEM((_BPW * _L,), jnp.float32),
        pltpu.VMEM((_L,), jnp.float32),
        pltpu.SemaphoreType.DMA,
    ],
)(_sc_body)


_N_TBL = 1000000
_TBLK = 512


def _transpose_body(in_ref, out_ref):
    out_ref[...] = in_ref[...].T


_transpose = pl.pallas_call(
    _transpose_body,
    grid=(pl.cdiv(_N_TBL, _TBLK),),
    in_specs=[pl.BlockSpec((_D, _TBLK), lambda g: (0, g))],
    out_specs=pl.BlockSpec((_TBLK, _D), lambda g: (g, 0)),
    out_shape=jax.ShapeDtypeStruct((_N_TBL, _D), jnp.float32),
)


def _finish_body(xp_ref, regp_ref, loss_ref, reg_ref):
    xp = xp_ref[...]
    grp = lax.broadcasted_iota(jnp.int32, (128, 8), 0) // _L
    col = lax.broadcasted_iota(jnp.int32, (128, 8), 1)
    diff = (grp - col).astype(jnp.float32)
    sel = 1.0 - jnp.abs(jnp.sign(diff))
    x = lax.dot_general(xp, sel, (((1,), (0,)), ((), ())),
                        preferred_element_type=jnp.float32)
    ls = jnp.minimum(x, 0.0) - jnp.log1p(jnp.exp(-jnp.abs(x)))
    loss_ref[...] = jnp.reshape(-jnp.sum(ls) * (1.0 / _BATCH), (1, 1))
    reg_ref[...] = jnp.reshape(jnp.sum(regp_ref[...]) * (1.0 / _BATCH), (1, 1))


_finish = pl.pallas_call(
    _finish_body,
    out_shape=(
        jax.ShapeDtypeStruct((1, 1), jnp.float32),
        jax.ShapeDtypeStruct((1, 1), jnp.float32),
    ),
)


def kernel(users, pos, neg, user_table, item_table):
    u_rm = _transpose(user_table.T)
    ut3 = u_rm.reshape(-1, _TPB, _D)
    it3 = item_table.reshape(-1, _TPB, _D)
    xp, regp = _sc_gather_dot(users, pos, neg, ut3, it3)
    loss, reg = _finish(xp.reshape(_BATCH // 8, 128), regp.reshape(4, 128))
    return loss.reshape(()), reg.reshape(())


# speedup vs baseline: 2.5556x; 2.5556x over previous
import functools

import jax
import jax.numpy as jnp
from jax import lax
from jax.experimental import pallas as pl
from jax.experimental.pallas import tpu as pltpu
from jax.experimental.pallas import tpu_sc as plsc

_BATCH = 16384
_D = 64
_TPB = 8
_NC = 2
_NS = 16
_NW = _NC * _NS
_BPW = _BATCH // _NW
_L = 16
_CH = 32
_NCH = _BPW // _CH


def _sc_body(users_h, pos_h, neg_h, ut_h, it_h,
             xp_out, reg_out,
             sdx_u, sdx_p, sdx_n,
             ru, rp, rn, parts, racc, sem):
    wid = lax.axis_index("s") * _NC + lax.axis_index("c")
    base = wid * _BPW

    for j in range(_BPW // 128):
        sl = pl.ds(base + j * 128, 128)
        pltpu.sync_copy(users_h.at[sl], sdx_u.at[j])
        pltpu.sync_copy(pos_h.at[sl], sdx_p.at[j])
        pltpu.sync_copy(neg_h.at[sl], sdx_n.at[j])

    zero = jnp.zeros((_L,), jnp.float32)

    def round_body(c, sacc):
        jr = lax.shift_right_logical(c, 2)
        orow = jnp.bitwise_and(c, 3) * _CH
        subs = []
        copies = []
        for g in range(_CH // _L):
            goff = orow + g * _L
            vu = sdx_u[jr, pl.ds(goff, _L)]
            vp = sdx_p[jr, pl.ds(goff, _L)]
            vn = sdx_n[jr, pl.ds(goff, _L)]
            for r in range(_L):
                i = g * _L + r
                eu, ep, en = vu[r], vp[r], vn[r]
                subs.append((jnp.bitwise_and(eu, 7), jnp.bitwise_and(ep, 7),
                             jnp.bitwise_and(en, 7)))
                copies.append(pltpu.async_copy(
                    ut_h.at[lax.shift_right_logical(eu, 3)], ru.at[i], sem))
                copies.append(pltpu.async_copy(
                    it_h.at[lax.shift_right_logical(ep, 3)], rp.at[i], sem))
                copies.append(pltpu.async_copy(
                    it_h.at[lax.shift_right_logical(en, 3)], rn.at[i], sem))
        for cp_ in copies:
            cp_.wait()
        for i in range(_CH):
            su, sp, sn = subs[i]
            pv = zero
            for kk in range(_D // _L):
                sl = pl.ds(kk * _L, _L)
                u = ru[i, su, sl]
                p = rp[i, sp, sl]
                n = rn[i, sn, sl]
                pv = pv + u * (p - n)
                sacc = sacc + u * u + p * p + n * n
            parts[pl.ds((c * _CH + i) * _L, _L)] = pv
        return sacc

    sacc = lax.fori_loop(0, _NCH, round_body, zero)
    racc[...] = sacc
    pltpu.sync_copy(parts, xp_out.at[pl.ds(base * _L, _BPW * _L)])
    pltpu.sync_copy(racc, reg_out.at[pl.ds(wid * _L, _L)])


_sc_gather_dot = functools.partial(
    pl.kernel,
    mesh=plsc.VectorSubcoreMesh(core_axis_name="c", subcore_axis_name="s"),
    out_type=[
        jax.ShapeDtypeStruct((_BATCH * _L,), jnp.float32),
        jax.ShapeDtypeStruct((_NW * _L,), jnp.float32),
    ],
    scratch_types=[
        pltpu.VMEM((_BPW // 128, 128), jnp.int32),
        pltpu.VMEM((_BPW // 128, 128), jnp.int32),
        pltpu.VMEM((_BPW // 128, 128), jnp.int32),
        pltpu.VMEM((_CH, _TPB, _D), jnp.float32),
        pltpu.VMEM((_CH, _TPB, _D), jnp.float32),
        pltpu.VMEM((_CH, _TPB, _D), jnp.float32),
        pltpu.VMEM((_BPW * _L,), jnp.float32),
        pltpu.VMEM((_L,), jnp.float32),
        pltpu.SemaphoreType.DMA,
    ],
)(_sc_body)


_N_TBL = 1000000
_TBLK = 8192


def _transpose_body(in_ref, out_ref):
    out_ref[...] = in_ref[...].T


_transpose = pl.pallas_call(
    _transpose_body,
    grid=(pl.cdiv(_N_TBL, _TBLK),),
    in_specs=[pl.BlockSpec((_D, _TBLK), lambda g: (0, g))],
    out_specs=pl.BlockSpec((_TBLK, _D), lambda g: (g, 0)),
    out_shape=jax.ShapeDtypeStruct((_N_TBL, _D), jnp.float32),
)


def _finish_body(xp_ref, regp_ref, loss_ref, reg_ref):
    xp = xp_ref[...]
    grp = lax.broadcasted_iota(jnp.int32, (128, 8), 0) // _L
    col = lax.broadcasted_iota(jnp.int32, (128, 8), 1)
    diff = (grp - col).astype(jnp.float32)
    sel = 1.0 - jnp.abs(jnp.sign(diff))
    x = lax.dot_general(xp, sel, (((1,), (0,)), ((), ())),
                        preferred_element_type=jnp.float32)
    ls = jnp.minimum(x, 0.0) - jnp.log1p(jnp.exp(-jnp.abs(x)))
    loss_ref[...] = jnp.reshape(-jnp.sum(ls) * (1.0 / _BATCH), (1, 1))
    reg_ref[...] = jnp.reshape(jnp.sum(regp_ref[...]) * (1.0 / _BATCH), (1, 1))


_finish = pl.pallas_call(
    _finish_body,
    out_shape=(
        jax.ShapeDtypeStruct((1, 1), jnp.float32),
        jax.ShapeDtypeStruct((1, 1), jnp.float32),
    ),
)


def kernel(users, pos, neg, user_table, item_table):
    u_rm = _transpose(user_table.T)
    ut3 = u_rm.reshape(-1, _TPB, _D)
    it3 = item_table.reshape(-1, _TPB, _D)
    xp, regp = _sc_gather_dot(users, pos, neg, ut3, it3)
    loss, reg = _finish(xp.reshape(_BATCH // 8, 128), regp.reshape(4, 128))
    return loss.reshape(()), reg.reshape(())
